# Initial kernel scaffold; baseline (speedup 1.0000x reference)
import jax
import jax.numpy as jnp
from jax.experimental import pallas as pl


def kernel(pred_logits, pred_boxes, target_sizes, target_start):
    raise NotImplementedError("write your pallas kernel here")


# speedup vs baseline: 1.0715x; 1.0715x over previous
import jax
import jax.numpy as jnp
from jax import lax
from jax.experimental import pallas as pl
from jax.experimental.pallas import tpu as pltpu
from jax.experimental.pallas import tpu_sc as plsc

B, Q, C = 16, 5000, 91
N = Q * C
K = 5
LANES = 16

HALF_V = 14219
HALF_E = HALF_V * LANES
START1 = N - HALF_E
CH_E = 16384
N_FULL = 13
REM_E = HALF_E - N_FULL * CH_E
_SIZES = [CH_E] * N_FULL + [REM_E]
_OFFS = [c * CH_E for c in range(N_FULL + 1)]

NEG_INF = float("-inf")
I32_BIG = 2147483647


def _scan_body(logits, cand_val, cand_idx, buf0, buf1, stage_v, stage_i,
               sem0, sem1):
    cid = lax.axis_index("c")
    sid = lax.axis_index("s")
    wid = sid * 2 + cid
    row = wid // 2
    start = (wid % 2) * START1
    gbase = row * N + start
    lane = lax.iota(jnp.int32, 16)

    sems = [sem0, sem1]
    bufs = [buf0, buf1]
    cps = [None, None]
    cps[0] = pltpu.async_copy(
        logits.at[pl.ds(pl.multiple_of(gbase, 8), CH_E)],
        buf0.at[pl.ds(0, CH_E)], sem0)

    def make_body(parity, base):
        def body(v, carry):
            rv = list(carry[:K])
            ri = list(carry[K:])
            x = bufs[parity][pl.ds(pl.multiple_of(v * 16, 8), 16)]
            xi = lane + (base + v * 16)
            for j in range(K):
                gt = x > rv[j]
                new_v = jnp.where(gt, x, rv[j])
                new_i = jnp.where(gt, xi, ri[j])
                x = jnp.where(gt, rv[j], x)
                xi = jnp.where(gt, ri[j], xi)
                rv[j] = new_v
                ri[j] = new_i
            return (*rv, *ri)
        return body

    carry = tuple([jnp.full((16,), NEG_INF, jnp.float32)] * K
                  + [jnp.zeros((16,), jnp.int32)] * K)
    for c in range(N_FULL + 1):
        p = c % 2
        cps[p].wait()
        if c + 1 <= N_FULL:
            q = (c + 1) % 2
            sz = _SIZES[c + 1]
            cps[q] = pltpu.async_copy(
                logits.at[pl.ds(pl.multiple_of(gbase + _OFFS[c + 1], 8), sz)],
                bufs[q].at[pl.ds(0, sz)], sems[q])
        carry = lax.fori_loop(0, _SIZES[c] // 16,
                              make_body(p, start + _OFFS[c]), carry)

    for j in range(K):
        stage_v[pl.ds(16 * j, 16)] = carry[j]
        stage_i[pl.ds(16 * j, 16)] = carry[K + j]
    pltpu.sync_copy(stage_v, cand_val.at[pl.ds(pl.multiple_of(80 * wid, 8), 80)])
    pltpu.sync_copy(stage_i, cand_idx.at[pl.ds(pl.multiple_of(80 * wid, 8), 80)])


_scan = pl.kernel(
    _scan_body,
    out_type=[
        jax.ShapeDtypeStruct((32 * 80,), jnp.float32),
        jax.ShapeDtypeStruct((32 * 80,), jnp.int32),
    ],
    mesh=plsc.VectorSubcoreMesh(core_axis_name="c", subcore_axis_name="s"),
    compiler_params=pltpu.CompilerParams(needs_layout_passes=False),
    scratch_types=[
        pltpu.VMEM((CH_E,), jnp.float32),
        pltpu.VMEM((CH_E,), jnp.float32),
        pltpu.VMEM((80,), jnp.float32),
        pltpu.VMEM((80,), jnp.int32),
        pltpu.SemaphoreType.DMA,
        pltpu.SemaphoreType.DMA,
    ],
)


def _merge_body(cand_val, cand_idx, pboxes, tsizes, tstart,
                scores_o, labels_o, boxes_o,
                cv, ci, boxrow, tsz, tst, ssc, lsc, bsc):
    cid = lax.axis_index("c")
    sid = lax.axis_index("s")
    wid = sid * 2 + cid

    @pl.when(wid < B)
    def _():
        r = wid
        pltpu.sync_copy(cand_val.at[pl.ds(pl.multiple_of(160 * r, 8), 160)], cv)
        pltpu.sync_copy(cand_idx.at[pl.ds(pl.multiple_of(160 * r, 8), 160)], ci)
        pltpu.sync_copy(pboxes.at[pl.ds(pl.multiple_of(4 * Q * r, 8), 4 * Q)],
                        boxrow)
        pltpu.sync_copy(tsizes, tsz)
        pltpu.sync_copy(tstart, tst)

        lane = lax.iota(jnp.int32, 16)
        vals = [cv[pl.ds(16 * k, 16)] for k in range(2 * K)]
        idxs = [ci[pl.ds(16 * k, 16)] for k in range(2 * K)]

        rsp = jnp.full((16,), r, jnp.int32)
        scale_v = plsc.load_gather(tsz, [2 * rsp + (lane + 1) % 2])
        start_v = plsc.load_gather(tst, [4 * rsp + lane % 2])
        sgn = jnp.where(lane % 4 < 2, jnp.float32(-0.5), jnp.float32(0.5))

        scores_reg = jnp.zeros((16,), jnp.float32)
        labels_reg = jnp.zeros((16,), jnp.int32)
        neg = jnp.full((16,), NEG_INF, jnp.float32)
        big = jnp.full((16,), I32_BIG, jnp.int32)

        for j in range(K):
            m = vals[0]
            for k in range(1, 2 * K):
                m = jnp.maximum(m, vals[k])
            msp = jnp.full((16,), jnp.max(m), jnp.float32)
            cidx = big
            for k in range(2 * K):
                cidx = jnp.minimum(cidx, jnp.where(vals[k] == msp, idxs[k], big))
            csp = jnp.full((16,), jnp.min(cidx), jnp.int32)
            for k in range(2 * K):
                vals[k] = jnp.where(idxs[k] == csp, neg, vals[k])

            sig = 1.0 / (1.0 + jnp.exp(-msp))
            scores_reg = jnp.where(lane == j, sig, scores_reg)
            labels_reg = jnp.where(lane == j, csp % C, labels_reg)

            bsp = csp // C
            a = plsc.load_gather(boxrow, [4 * bsp + lane % 2])
            b = plsc.load_gather(boxrow, [4 * bsp + lane % 2 + 2])
            res = (a + sgn * b + start_v) * scale_v
            plsc.store_scatter(bsc, [8 * j + lane % 4], res, mask=lane < 4)

        ssc[...] = scores_reg
        lsc[...] = labels_reg
        pltpu.sync_copy(ssc.at[pl.ds(0, 8)],
                        scores_o.at[pl.ds(pl.multiple_of(8 * r, 8), 8)])
        pltpu.sync_copy(lsc.at[pl.ds(0, 8)],
                        labels_o.at[pl.ds(pl.multiple_of(8 * r, 8), 8)])
        pltpu.sync_copy(bsc, boxes_o.at[pl.ds(pl.multiple_of(64 * r, 8), 64)])


_merge = pl.kernel(
    _merge_body,
    out_type=[
        jax.ShapeDtypeStruct((B * 8,), jnp.float32),
        jax.ShapeDtypeStruct((B * 8,), jnp.int32),
        jax.ShapeDtypeStruct((B * 64,), jnp.float32),
    ],
    mesh=plsc.VectorSubcoreMesh(core_axis_name="c", subcore_axis_name="s"),
    compiler_params=pltpu.CompilerParams(needs_layout_passes=False),
    scratch_types=[
        pltpu.VMEM((160,), jnp.float32),
        pltpu.VMEM((160,), jnp.int32),
        pltpu.VMEM((4 * Q,), jnp.float32),
        pltpu.VMEM((2 * B,), jnp.float32),
        pltpu.VMEM((4 * B,), jnp.float32),
        pltpu.VMEM((16,), jnp.float32),
        pltpu.VMEM((16,), jnp.int32),
        pltpu.VMEM((64,), jnp.float32),
    ],
)


def kernel(pred_logits, pred_boxes, target_sizes, target_start):
    flat = pred_logits.reshape(B * N)
    cand_v, cand_i = _scan(flat)
    scores_p, labels_p, boxes_p = _merge(
        cand_v, cand_i, pred_boxes.reshape(-1), target_sizes.reshape(-1),
        target_start.reshape(-1))
    scores = scores_p.reshape(B, 8)[:, :K]
    labels = labels_p.reshape(B, 8)[:, :K]
    boxes = boxes_p.reshape(B, 8, 8)[:, :K, :4]
    return scores, labels, boxes


# speedup vs baseline: 2.1083x; 1.9677x over previous
import jax
import jax.numpy as jnp
from jax import lax
from jax.experimental import pallas as pl
from jax.experimental.pallas import tpu as pltpu
from jax.experimental.pallas import tpu_sc as plsc

B, Q, C = 16, 5000, 91
N = Q * C
K = 5
LANES = 16

QR = 2504
Q1 = Q - QR
CH_LIST = [312] * 8 + [8]
_QOFFS = [sum(CH_LIST[:c]) for c in range(len(CH_LIST) + 1)]
C0S = [0, 16, 32, 48, 64, 75]

NEG_INF = float("-inf")
I32_BIG = 2147483647


def _scan_body(logits, cand_val, cand_idx, buf0, buf1, stage_v, stage_i,
               sem0, sem1):
    cid = lax.axis_index("c")
    sid = lax.axis_index("s")
    wid = sid * 2 + cid
    row = wid // 2
    qbase = (wid % 2) * Q1
    lane = lax.iota(jnp.int32, 16)

    sems = [sem0, sem1]
    bufs = [buf0, buf1]
    cps = [None, None]

    def start_dma(c, p):
        q0 = pl.multiple_of(qbase + _QOFFS[c], 8)
        sz = CH_LIST[c]
        return pltpu.async_copy(logits.at[row, pl.ds(q0, sz), :],
                                bufs[p].at[pl.ds(0, sz), :], sems[p])

    cps[0] = start_dma(0, 0)

    def make_body(parity, chunk_q0):
        buf = bufs[parity]

        def body(q, carry):
            rv = list(carry[:K])
            ri = list(carry[K:])
            cbase = (chunk_q0 + q) * C
            for c0 in C0S:
                x = buf[q, pl.ds(c0, 16)]
                xi = lane + (cbase + c0)
                for j in range(K):
                    gt = x > rv[j]
                    new_v = jnp.where(gt, x, rv[j])
                    new_i = jnp.where(gt, xi, ri[j])
                    x = jnp.where(gt, rv[j], x)
                    xi = jnp.where(gt, ri[j], xi)
                    rv[j] = new_v
                    ri[j] = new_i
            return (*rv, *ri)
        return body

    carry = tuple([jnp.full((16,), NEG_INF, jnp.float32)] * K
                  + [jnp.zeros((16,), jnp.int32)] * K)
    for c in range(len(CH_LIST)):
        p = c % 2
        cps[p].wait()
        if c + 1 < len(CH_LIST):
            cps[(c + 1) % 2] = start_dma(c + 1, (c + 1) % 2)
        carry = lax.fori_loop(0, CH_LIST[c],
                              make_body(p, qbase + _QOFFS[c]), carry)

    for j in range(K):
        stage_v[0, pl.ds(16 * j, 16)] = carry[j]
        stage_i[0, pl.ds(16 * j, 16)] = carry[K + j]
    pltpu.sync_copy(stage_v, cand_val.at[wid])
    pltpu.sync_copy(stage_i, cand_idx.at[wid])


_scan = pl.kernel(
    _scan_body,
    out_type=[
        jax.ShapeDtypeStruct((32, 1, 80), jnp.float32),
        jax.ShapeDtypeStruct((32, 1, 80), jnp.int32),
    ],
    mesh=plsc.VectorSubcoreMesh(core_axis_name="c", subcore_axis_name="s"),
    scratch_types=[
        pltpu.VMEM((CH_LIST[0], C), jnp.float32),
        pltpu.VMEM((CH_LIST[0], C), jnp.float32),
        pltpu.VMEM((1, 80), jnp.float32),
        pltpu.VMEM((1, 80), jnp.int32),
        pltpu.SemaphoreType.DMA,
        pltpu.SemaphoreType.DMA,
    ],
)


def _merge_body(cand_val, cand_idx, pboxes, tsizes, tstart,
                scores_o, labels_o, boxes_o,
                cv, ci, boxrow, tsz, tst, ssc, lsc, bsc):
    cid = lax.axis_index("c")
    sid = lax.axis_index("s")
    wid = sid * 2 + cid

    @pl.when(wid < B)
    def _():
        r = wid
        pltpu.sync_copy(cand_val.at[pl.ds(2 * r, 2)], cv)
        pltpu.sync_copy(cand_idx.at[pl.ds(2 * r, 2)], ci)
        pltpu.sync_copy(pboxes.at[pl.ds(pl.multiple_of(4 * Q * r, 8), 4 * Q)],
                        boxrow)
        pltpu.sync_copy(tsizes, tsz)
        pltpu.sync_copy(tstart, tst)

        lane = lax.iota(jnp.int32, 16)
        vals = [cv[k // K, 0, pl.ds(16 * (k % K), 16)] for k in range(2 * K)]
        idxs = [ci[k // K, 0, pl.ds(16 * (k % K), 16)] for k in range(2 * K)]

        rsp = jnp.full((16,), r, jnp.int32)
        scale_v = plsc.load_gather(tsz, [2 * rsp + (lane + 1) % 2])
        start_v = plsc.load_gather(tst, [4 * rsp + lane % 2])
        sgn = jnp.where(lane % 4 < 2, jnp.float32(-0.5), jnp.float32(0.5))

        scores_reg = jnp.zeros((16,), jnp.float32)
        labels_reg = jnp.zeros((16,), jnp.int32)
        neg = jnp.full((16,), NEG_INF, jnp.float32)
        big = jnp.full((16,), I32_BIG, jnp.int32)

        for j in range(K):
            m = vals[0]
            for k in range(1, 2 * K):
                m = jnp.maximum(m, vals[k])
            msp = jnp.full((16,), jnp.max(m), jnp.float32)
            cidx = big
            for k in range(2 * K):
                cidx = jnp.minimum(cidx, jnp.where(vals[k] == msp, idxs[k], big))
            csp = jnp.full((16,), jnp.min(cidx), jnp.int32)
            for k in range(2 * K):
                vals[k] = jnp.where(idxs[k] == csp, neg, vals[k])

            sig = 1.0 / (1.0 + jnp.exp(-msp))
            scores_reg = jnp.where(lane == j, sig, scores_reg)
            labels_reg = jnp.where(lane == j, csp % C, labels_reg)

            bsp = csp // C
            a = plsc.load_gather(boxrow, [4 * bsp + lane % 2])
            b = plsc.load_gather(boxrow, [4 * bsp + lane % 2 + 2])
            res = (a + sgn * b + start_v) * scale_v
            plsc.store_scatter(bsc, [8 * j + lane % 4], res, mask=lane < 4)

        ssc[...] = scores_reg
        lsc[...] = labels_reg
        pltpu.sync_copy(ssc.at[pl.ds(0, 8)],
                        scores_o.at[pl.ds(pl.multiple_of(8 * r, 8), 8)])
        pltpu.sync_copy(lsc.at[pl.ds(0, 8)],
                        labels_o.at[pl.ds(pl.multiple_of(8 * r, 8), 8)])
        pltpu.sync_copy(bsc, boxes_o.at[pl.ds(pl.multiple_of(64 * r, 8), 64)])


_merge = pl.kernel(
    _merge_body,
    out_type=[
        jax.ShapeDtypeStruct((B * 8,), jnp.float32),
        jax.ShapeDtypeStruct((B * 8,), jnp.int32),
        jax.ShapeDtypeStruct((B * 64,), jnp.float32),
    ],
    mesh=plsc.VectorSubcoreMesh(core_axis_name="c", subcore_axis_name="s"),
    compiler_params=pltpu.CompilerParams(needs_layout_passes=False),
    scratch_types=[
        pltpu.VMEM((2, 1, 80), jnp.float32),
        pltpu.VMEM((2, 1, 80), jnp.int32),
        pltpu.VMEM((4 * Q,), jnp.float32),
        pltpu.VMEM((2 * B,), jnp.float32),
        pltpu.VMEM((4 * B,), jnp.float32),
        pltpu.VMEM((16,), jnp.float32),
        pltpu.VMEM((16,), jnp.int32),
        pltpu.VMEM((64,), jnp.float32),
    ],
)


def kernel(pred_logits, pred_boxes, target_sizes, target_start):
    cand_v, cand_i = _scan(pred_logits)
    scores_p, labels_p, boxes_p = _merge(
        cand_v, cand_i, pred_boxes.reshape(-1), target_sizes.reshape(-1),
        target_start.reshape(-1))
    scores = scores_p.reshape(B, 8)[:, :K]
    labels = labels_p.reshape(B, 8)[:, :K]
    boxes = boxes_p.reshape(B, 8, 8)[:, :K, :4]
    return scores, labels, boxes
